# Initial kernel scaffold; baseline (speedup 1.0000x reference)
import functools

import jax
import jax.numpy as jnp
from jax import lax
from jax.experimental import pallas as pl
from jax.experimental.pallas import tpu as pltpu
from jax.experimental.pallas import tpu_sc as plsc

CH = 40
LANES = 16


def _tc_proj(h, bcat, wself, blk=400):
    n, din = h.shape
    dcat = bcat.shape[1]
    dout = wself.shape[1]

    def body(h_ref, bcat_ref, ws_ref, hb_ref, self_ref):
        hblk = h_ref[...]
        hb_ref[...] = jnp.dot(hblk, bcat_ref[...], preferred_element_type=jnp.float32)
        self_ref[...] = jnp.dot(hblk, ws_ref[...], preferred_element_type=jnp.float32)

    return pl.pallas_call(
        body,
        grid=(n // blk,),
        in_specs=[
            pl.BlockSpec((blk, din), lambda m: (m, 0)),
            pl.BlockSpec((din, dcat), lambda m: (0, 0)),
            pl.BlockSpec((din, dout), lambda m: (0, 0)),
        ],
        out_specs=[
            pl.BlockSpec((blk, dcat), lambda m: (m, 0)),
            pl.BlockSpec((blk, dout), lambda m: (m, 0)),
        ],
        out_shape=[
            jax.ShapeDtypeStruct((n, dcat), jnp.float32),
            jax.ShapeDtypeStruct((n, dout), jnp.float32),
        ],
    )(h, bcat, wself)


def _tc_mid(aggp, degp, selfp, bias, bcat, wself, blk=400):
    n, din = selfp.shape
    dcat = bcat.shape[1]
    dout = wself.shape[1]

    def body(agg_ref, deg_ref, self_ref, bias_ref, bcat_ref, ws_ref, hb_ref, self_out_ref):
        agg = agg_ref[0] + agg_ref[1]
        deg = jnp.maximum(deg_ref[0, :, 0:1] + deg_ref[1, :, 0:1], 1.0)
        pre = agg / deg + self_ref[...] + bias_ref[...]
        h1 = jnp.where(pre > 0, pre, 0.2 * pre)
        hb_ref[...] = jnp.dot(h1, bcat_ref[...], preferred_element_type=jnp.float32)
        self_out_ref[...] = jnp.dot(h1, ws_ref[...], preferred_element_type=jnp.float32)

    return pl.pallas_call(
        body,
        grid=(n // blk,),
        in_specs=[
            pl.BlockSpec((2, blk, din), lambda m: (0, m, 0)),
            pl.BlockSpec((2, blk, LANES), lambda m: (0, m, 0)),
            pl.BlockSpec((blk, din), lambda m: (m, 0)),
            pl.BlockSpec((1, din), lambda m: (0, 0)),
            pl.BlockSpec((din, dcat), lambda m: (0, 0)),
            pl.BlockSpec((din, dout), lambda m: (0, 0)),
        ],
        out_specs=[
            pl.BlockSpec((blk, dcat), lambda m: (m, 0)),
            pl.BlockSpec((blk, dout), lambda m: (m, 0)),
        ],
        out_shape=[
            jax.ShapeDtypeStruct((n, dcat), jnp.float32),
            jax.ShapeDtypeStruct((n, dout), jnp.float32),
        ],
    )(aggp, degp, selfp, bias, bcat, wself)


def _tc_final(aggp, degp, selfp, bias, blk=400):
    n, dout = selfp.shape

    def body(agg_ref, deg_ref, self_ref, bias_ref, out_ref):
        agg = agg_ref[0] + agg_ref[1]
        deg = jnp.maximum(deg_ref[0, :, 0:1] + deg_ref[1, :, 0:1], 1.0)
        out_ref[...] = agg / deg + self_ref[...] + bias_ref[...]

    return pl.pallas_call(
        body,
        grid=(n // blk,),
        in_specs=[
            pl.BlockSpec((2, blk, dout), lambda m: (0, m, 0)),
            pl.BlockSpec((2, blk, LANES), lambda m: (0, m, 0)),
            pl.BlockSpec((blk, dout), lambda m: (m, 0)),
            pl.BlockSpec((1, dout), lambda m: (0, 0)),
        ],
        out_specs=pl.BlockSpec((blk, dout), lambda m: (m, 0)),
        out_shape=jax.ShapeDtypeStruct((n, dout), jnp.float32),
    )(aggp, degp, selfp, bias)


@functools.lru_cache(maxsize=None)
def _make_sc_edge(n_nodes, e_chunks, nb, dout, compute_deg):
    info = plsc.get_sparse_core_info()
    nc, ns = info.num_cores, info.num_subcores
    nw = nc * ns
    chunks_w = e_chunks // nw
    rows_t = n_nodes // ns
    nj = dout // LANES
    mesh = plsc.VectorSubcoreMesh(core_axis_name="c", subcore_axis_name="s")

    out_type = [jax.ShapeDtypeStruct((nc * n_nodes, dout), jnp.float32)]
    if compute_deg:
        out_type.append(jax.ShapeDtypeStruct((nc * n_nodes, LANES), jnp.float32))

    scratch = [
        pltpu.VMEM((chunks_w, CH), jnp.int32),
        pltpu.VMEM((chunks_w, CH), jnp.int32),
        pltpu.VMEM((chunks_w, CH), jnp.int32),
        pltpu.VMEM((CH, LANES), jnp.float32),
        pltpu.VMEM((CH, nb * dout), jnp.float32),
        pltpu.VMEM((CH, dout), jnp.float32),
        pltpu.VMEM_SHARED((n_nodes, dout), jnp.float32),
        pltpu.SemaphoreType.DMA,
        pltpu.SemaphoreType.DMA,
    ]
    if compute_deg:
        scratch.append(pltpu.VMEM((CH, LANES), jnp.float32))
        scratch.append(pltpu.VMEM_SHARED((n_nodes, LANES), jnp.float32))

    def body(src_hbm, rel_hbm, dst_hbm, hb_hbm, coeff_hbm, zero_hbm, zerol_hbm,
             *rest):
        if compute_deg:
            agg_out, deg_out = rest[0], rest[1]
            (srcv, relv, dstv, cbuf, rows, msg, aggsh, sem1, sem2,
             onesv, degsh) = rest[2:]
        else:
            agg_out = rest[0]
            (srcv, relv, dstv, cbuf, rows, msg, aggsh, sem1, sem2) = rest[1:]
        cid = lax.axis_index("c")
        sid = lax.axis_index("s")
        wid = sid * nc + cid
        r0 = sid * rows_t

        pltpu.sync_copy(zero_hbm.at[pl.ds(r0, rows_t)], aggsh.at[pl.ds(r0, rows_t)])
        if compute_deg:
            pltpu.sync_copy(zerol_hbm.at[pl.ds(r0, rows_t)], degsh.at[pl.ds(r0, rows_t)])
            one = jnp.ones((LANES,), jnp.float32)
            for i in range(CH):
                onesv[i, :] = one
        c0 = wid * chunks_w
        pltpu.sync_copy(src_hbm.at[pl.ds(c0, chunks_w)], srcv)
        pltpu.sync_copy(rel_hbm.at[pl.ds(c0, chunks_w)], relv)
        pltpu.sync_copy(dst_hbm.at[pl.ds(c0, chunks_w)], dstv)
        plsc.subcore_barrier()

        def chunk_body(c, carry):
            pltpu.async_copy(coeff_hbm.at[relv.at[c]], cbuf, sem1).wait()
            pltpu.async_copy(hb_hbm.at[srcv.at[c]], rows, sem2).wait()

            def edge_body(i, carry2):
                accs = [jnp.zeros((LANES,), jnp.float32) for _ in range(nj)]
                for b in range(nb):
                    cb = cbuf[i, b]
                    for j in range(nj):
                        accs[j] = accs[j] + cb * rows[i, pl.ds(b * dout + j * LANES, LANES)]
                for j in range(nj):
                    msg[i, pl.ds(j * LANES, LANES)] = accs[j]
                return carry2

            lax.fori_loop(0, CH, edge_body, 0)
            pltpu.sync_copy(msg, aggsh.at[dstv.at[c]], add=True)
            if compute_deg:
                pltpu.sync_copy(onesv, degsh.at[dstv.at[c]], add=True)
            return carry

        lax.fori_loop(0, chunks_w, chunk_body, 0)
        plsc.subcore_barrier()

        pltpu.sync_copy(aggsh.at[pl.ds(r0, rows_t)],
                        agg_out.at[pl.ds(cid * n_nodes + r0, rows_t)])
        if compute_deg:
            @pl.when(sid == 0)
            def _():
                pltpu.sync_copy(degsh, deg_out.at[pl.ds(cid * n_nodes, n_nodes)])

    return pl.kernel(body, out_type=out_type, mesh=mesh, scratch_types=scratch)


def _rgcn(edges3, entity_embed, bases, coeff, wself, bias):
    n, din = entity_embed.shape
    nb = bases[0].shape[0]
    info = plsc.get_sparse_core_info()
    nc = info.num_cores

    src2, rel2, dst2 = edges3
    e_chunks = src2.shape[0]
    zero_hbm = jnp.zeros((n, din), jnp.float32)
    zerol_hbm = jnp.zeros((n, LANES), jnp.float32)

    d1 = bases[0].shape[2]
    d2 = bases[1].shape[2]
    bcat0 = jnp.transpose(bases[0], (1, 0, 2)).reshape(din, nb * d1)
    bcat1 = jnp.transpose(bases[1], (1, 0, 2)).reshape(d1, nb * d2)
    cpad0 = jnp.pad(coeff[0], ((0, 0), (0, LANES - nb)))
    cpad1 = jnp.pad(coeff[1], ((0, 0), (0, LANES - nb)))

    hb0, self0 = _tc_proj(entity_embed, bcat0, wself[0])
    sc0 = _make_sc_edge(n, e_chunks, nb, d1, True)
    agg0f, deg0f = sc0(src2, rel2, dst2, hb0, cpad0, zero_hbm, zerol_hbm)
    agg0 = agg0f.reshape(nc, n, d1)
    degp = deg0f.reshape(nc, n, LANES)
    hb1, self1 = _tc_mid(agg0, degp, self0, bias[0].reshape(1, -1), bcat1, wself[1])
    sc1 = _make_sc_edge(n, e_chunks, nb, d2, False)
    agg1f = sc1(src2, rel2, dst2, hb1, cpad1, zero_hbm, zerol_hbm)
    agg1 = agg1f.reshape(nc, n, d2)
    return _tc_final(agg1, degp, self1, bias[1].reshape(1, -1))


def kernel(edges, entity_embed, bases0, coeff0, wself0, bias0,
           bases1, coeff1, wself1, bias1):
    e = edges.shape[0]
    src2 = edges[:, 0].reshape(e // CH, CH)
    rel2 = edges[:, 1].reshape(e // CH, CH)
    dst2 = edges[:, 2].reshape(e // CH, CH)
    return _rgcn((src2, rel2, dst2), entity_embed,
                 (bases0, bases1), (coeff0, coeff1),
                 (wself0, wself1), (bias0, bias1))


# speedup vs baseline: 4.5232x; 4.5232x over previous
import functools

import jax
import jax.numpy as jnp
from jax import lax
from jax.experimental import pallas as pl
from jax.experimental.pallas import tpu as pltpu
from jax.experimental.pallas import tpu_sc as plsc

CH = 16
STAGE = 25
LANES = 16


def _tc_proj(h, bcat, wself, blk=400):
    n, din = h.shape
    dcat = bcat.shape[1]
    dout = wself.shape[1]

    def body(h_ref, bcat_ref, ws_ref, hb_ref, self_ref):
        hblk = h_ref[...]
        hb_ref[...] = jnp.dot(hblk, bcat_ref[...], preferred_element_type=jnp.float32)
        self_ref[...] = jnp.dot(hblk, ws_ref[...], preferred_element_type=jnp.float32)

    return pl.pallas_call(
        body,
        grid=(n // blk,),
        in_specs=[
            pl.BlockSpec((blk, din), lambda m: (m, 0)),
            pl.BlockSpec((din, dcat), lambda m: (0, 0)),
            pl.BlockSpec((din, dout), lambda m: (0, 0)),
        ],
        out_specs=[
            pl.BlockSpec((blk, dcat), lambda m: (m, 0)),
            pl.BlockSpec((blk, dout), lambda m: (m, 0)),
        ],
        out_shape=[
            jax.ShapeDtypeStruct((n, dcat), jnp.float32),
            jax.ShapeDtypeStruct((n, dout), jnp.float32),
        ],
    )(h, bcat, wself)


def _tc_mid(aggp, degp, selfp, bias, bcat, wself, blk=400):
    n, din = selfp.shape
    dcat = bcat.shape[1]
    dout = wself.shape[1]

    def body(agg_ref, deg_ref, self_ref, bias_ref, bcat_ref, ws_ref, hb_ref, self_out_ref):
        agg = agg_ref[0] + agg_ref[1]
        deg = jnp.maximum(deg_ref[0, :, 0:1] + deg_ref[1, :, 0:1], 1.0)
        pre = agg / deg + self_ref[...] + bias_ref[...]
        h1 = jnp.where(pre > 0, pre, 0.2 * pre)
        hb_ref[...] = jnp.dot(h1, bcat_ref[...], preferred_element_type=jnp.float32)
        self_out_ref[...] = jnp.dot(h1, ws_ref[...], preferred_element_type=jnp.float32)

    return pl.pallas_call(
        body,
        grid=(n // blk,),
        in_specs=[
            pl.BlockSpec((2, blk, din), lambda m: (0, m, 0)),
            pl.BlockSpec((2, blk, 128), lambda m: (0, m, 0)),
            pl.BlockSpec((blk, din), lambda m: (m, 0)),
            pl.BlockSpec((1, din), lambda m: (0, 0)),
            pl.BlockSpec((din, dcat), lambda m: (0, 0)),
            pl.BlockSpec((din, dout), lambda m: (0, 0)),
        ],
        out_specs=[
            pl.BlockSpec((blk, dcat), lambda m: (m, 0)),
            pl.BlockSpec((blk, dout), lambda m: (m, 0)),
        ],
        out_shape=[
            jax.ShapeDtypeStruct((n, dcat), jnp.float32),
            jax.ShapeDtypeStruct((n, dout), jnp.float32),
        ],
    )(aggp, degp, selfp, bias, bcat, wself)


def _tc_final(aggp, degp, selfp, bias, blk=400):
    n, dout = selfp.shape

    def body(agg_ref, deg_ref, self_ref, bias_ref, out_ref):
        agg = agg_ref[0] + agg_ref[1]
        deg = jnp.maximum(deg_ref[0, :, 0:1] + deg_ref[1, :, 0:1], 1.0)
        out_ref[...] = agg / deg + self_ref[...] + bias_ref[...]

    return pl.pallas_call(
        body,
        grid=(n // blk,),
        in_specs=[
            pl.BlockSpec((2, blk, dout), lambda m: (0, m, 0)),
            pl.BlockSpec((2, blk, 128), lambda m: (0, m, 0)),
            pl.BlockSpec((blk, dout), lambda m: (m, 0)),
            pl.BlockSpec((1, dout), lambda m: (0, 0)),
        ],
        out_specs=pl.BlockSpec((blk, dout), lambda m: (m, 0)),
        out_shape=jax.ShapeDtypeStruct((n, dout), jnp.float32),
    )(aggp, degp, selfp, bias)


@functools.lru_cache(maxsize=None)
def _make_sc_edge(n_pad, e_chunks, nb, dout):
    info = plsc.get_sparse_core_info()
    nc, ns = info.num_cores, info.num_subcores
    nw = nc * ns
    chunks_w = e_chunks // nw
    rows_t = n_pad // ns
    nj = dout // LANES
    mesh = plsc.VectorSubcoreMesh(core_axis_name="c", subcore_axis_name="s")

    out_type = [jax.ShapeDtypeStruct((nc * n_pad, dout), jnp.float32)]

    scratch = [
        pltpu.VMEM((STAGE, CH), jnp.int32),
        pltpu.VMEM((STAGE, CH), jnp.int32),
        pltpu.VMEM((STAGE, CH), jnp.int32),
        pltpu.VMEM((CH, 128), jnp.float32),
        pltpu.VMEM((CH, nb * dout), jnp.float32),
        pltpu.VMEM((CH, dout), jnp.float32),
        pltpu.VMEM_SHARED((n_pad, dout), jnp.float32),
        pltpu.SemaphoreType.DMA,
        pltpu.SemaphoreType.DMA,
    ]

    def body(src_hbm, rel_hbm, dst_hbm, hb_hbm, coeff_hbm, zero_hbm,
             agg_out, srcv, relv, dstv, cbuf, rows, msg, aggsh, sem1, sem2):
        cid = lax.axis_index("c")
        sid = lax.axis_index("s")
        wid = sid * nc + cid
        r0 = sid * rows_t

        pltpu.sync_copy(zero_hbm.at[pl.ds(r0, rows_t)], aggsh.at[pl.ds(r0, rows_t)])
        plsc.subcore_barrier()

        def block_body(bi, carry0):
            pltpu.sync_copy(src_hbm.at[wid, bi], srcv)
            pltpu.sync_copy(rel_hbm.at[wid, bi], relv)
            pltpu.sync_copy(dst_hbm.at[wid, bi], dstv)

            def chunk_body(c, carry):
                pltpu.async_copy(coeff_hbm.at[relv.at[c]], cbuf, sem1).wait()
                pltpu.async_copy(hb_hbm.at[srcv.at[c]], rows, sem2).wait()

                def edge_body(i, carry2):
                    accs = [jnp.zeros((LANES,), jnp.float32) for _ in range(nj)]
                    cvec = cbuf[i, pl.ds(0, LANES)]
                    for b in range(nb):
                        cb = cvec[b]
                        for j in range(nj):
                            accs[j] = accs[j] + cb * rows[i, pl.ds(b * dout + j * LANES, LANES)]
                    for j in range(nj):
                        msg[i, pl.ds(j * LANES, LANES)] = accs[j]
                    return carry2

                lax.fori_loop(0, CH, edge_body, 0)
                pltpu.sync_copy(msg, aggsh.at[dstv.at[c]], add=True)
                return carry

            lax.fori_loop(0, STAGE, chunk_body, 0)
            return carry0

        lax.fori_loop(0, chunks_w // STAGE, block_body, 0)
        plsc.subcore_barrier()

        pltpu.sync_copy(aggsh.at[pl.ds(r0, rows_t)],
                        agg_out.at[pl.ds(cid * n_pad + r0, rows_t)])

    return pl.kernel(body, out_type=out_type, mesh=mesh, scratch_types=scratch)


CHD = 100
STAGED = 25


@functools.lru_cache(maxsize=None)
def _make_sc_deg(n_pad):
    info = plsc.get_sparse_core_info()
    nc, ns = info.num_cores, info.num_subcores
    mesh = plsc.VectorSubcoreMesh(core_axis_name="c", subcore_axis_name="s")
    rows_t = n_pad // ns

    scratch = [
        pltpu.VMEM((STAGED, CHD), jnp.int32),
        pltpu.VMEM((CHD, 128), jnp.float32),
        pltpu.VMEM_SHARED((n_pad, 128), jnp.float32),
        pltpu.SemaphoreType.DMA,
    ]

    def body(dst_hbm, one_hbm, zero_hbm, deg_out, dstv, onesv, degsh, sem):
        cid = lax.axis_index("c")
        sid = lax.axis_index("s")
        wid = sid * nc + cid
        r0 = sid * rows_t
        nblk = dst_hbm.shape[1]
        pltpu.sync_copy(zero_hbm.at[pl.ds(r0, rows_t)], degsh.at[pl.ds(r0, rows_t)])
        pltpu.sync_copy(one_hbm, onesv)
        plsc.subcore_barrier()

        def block_body(bi, carry0):
            pltpu.sync_copy(dst_hbm.at[wid, bi], dstv)

            def fire(c, carry):
                pltpu.async_copy(onesv, degsh.at[dstv.at[c]], sem, add=True)
                return carry

            lax.fori_loop(0, STAGED, fire, 0)

            def drain(c, carry):
                pltpu.make_async_copy(onesv, degsh.at[dstv.at[0]], sem).wait()
                return carry

            lax.fori_loop(0, STAGED, drain, 0)
            return carry0

        lax.fori_loop(0, nblk, block_body, 0)
        plsc.subcore_barrier()
        pltpu.sync_copy(degsh.at[pl.ds(r0, rows_t)],
                        deg_out.at[pl.ds(cid * n_pad + r0, rows_t)])

    return pl.kernel(body,
                     out_type=[jax.ShapeDtypeStruct((nc * n_pad, 128), jnp.float32)],
                     mesh=mesh, scratch_types=scratch)


def _rgcn(edges3, entity_embed, bases, coeff, wself, bias):
    n, din = entity_embed.shape
    nb = bases[0].shape[0]
    info = plsc.get_sparse_core_info()
    nc, ns = info.num_cores, info.num_subcores
    rows_t = -(-n // ns // 8) * 8
    n_pad = rows_t * ns

    src2, rel2, dst2, dst3 = edges3
    e_chunks = src2.shape[0] * src2.shape[1] * src2.shape[2] * src2.shape[3] // CH
    zero_hbm = jnp.zeros((n_pad, din), jnp.float32)
    one_hbm = jnp.ones((CHD, 128), jnp.float32)

    d1 = bases[0].shape[2]
    d2 = bases[1].shape[2]
    bcat0 = jnp.transpose(bases[0], (1, 0, 2)).reshape(din, nb * d1)
    bcat1 = jnp.transpose(bases[1], (1, 0, 2)).reshape(d1, nb * d2)
    cpad0 = jnp.pad(coeff[0], ((0, 0), (0, 128 - nb)))
    cpad1 = jnp.pad(coeff[1], ((0, 0), (0, 128 - nb)))

    hb0, self0 = _tc_proj(entity_embed, bcat0, wself[0])
    (degf,) = _make_sc_deg(n_pad)(dst3, one_hbm, zero_hbm)
    degp = degf.reshape(nc, n_pad, 128)
    sc_edge = _make_sc_edge(n_pad, e_chunks, nb, d1)
    (agg0f,) = sc_edge(src2, rel2, dst2, hb0, cpad0, zero_hbm)
    agg0 = agg0f.reshape(nc, n_pad, d1)
    hb1, self1 = _tc_mid(agg0, degp, self0, bias[0].reshape(1, -1), bcat1, wself[1])
    (agg1f,) = sc_edge(src2, rel2, dst2, hb1, cpad1, zero_hbm)
    agg1 = agg1f.reshape(nc, n_pad, d2)
    return _tc_final(agg1, degp, self1, bias[1].reshape(1, -1))


def kernel(edges, entity_embed, bases0, coeff0, wself0, bias0,
           bases1, coeff1, wself1, bias1):
    e = edges.shape[0]
    info = plsc.get_sparse_core_info()
    nw = info.num_cores * info.num_subcores
    nblk = e // (nw * CH * STAGE)
    src2 = edges[:, 0].reshape(nw, nblk, STAGE, CH)
    rel2 = edges[:, 1].reshape(nw, nblk, STAGE, CH)
    dst2 = edges[:, 2].reshape(nw, nblk, STAGE, CH)
    nblkd = e // (nw * CHD * STAGED)
    dst3 = edges[:, 2].reshape(nw, nblkd, STAGED, CHD)
    return _rgcn((src2, rel2, dst2, dst3), entity_embed,
                 (bases0, bases1), (coeff0, coeff1),
                 (wself0, wself1), (bias0, bias1))


# speedup vs baseline: 11.2486x; 2.4868x over previous
import functools

import jax
import jax.numpy as jnp
from jax import lax
from jax.experimental import pallas as pl
from jax.experimental.pallas import tpu as pltpu
from jax.experimental.pallas import tpu_sc as plsc

CH = 16
STAGE = 25
LANES = 16


def _pack_bf16(lo, hi):
    lou = lax.bitcast_convert_type(lo, jnp.uint32)
    hiu = lax.bitcast_convert_type(hi, jnp.uint32)
    packed = ((hiu + jnp.uint32(0x8000)) & jnp.uint32(0xFFFF0000)) | (
        (lou + jnp.uint32(0x8000)) >> 16)
    return lax.bitcast_convert_type(packed, jnp.int32)


def _tc_proj(h, bcat_lo, bcat_hi, wself, blk=400):
    n, din = h.shape
    dcat = bcat_lo.shape[1]
    dout = wself.shape[1]

    def body(h_ref, blo_ref, bhi_ref, ws_ref, hbp_ref, self_ref):
        hblk = h_ref[...]
        lo = jnp.dot(hblk, blo_ref[...], preferred_element_type=jnp.float32)
        hi = jnp.dot(hblk, bhi_ref[...], preferred_element_type=jnp.float32)
        hbp_ref[...] = _pack_bf16(lo, hi)
        self_ref[...] = jnp.dot(hblk, ws_ref[...], preferred_element_type=jnp.float32)

    return pl.pallas_call(
        body,
        grid=(n // blk,),
        in_specs=[
            pl.BlockSpec((blk, din), lambda m: (m, 0)),
            pl.BlockSpec((din, dcat), lambda m: (0, 0)),
            pl.BlockSpec((din, dcat), lambda m: (0, 0)),
            pl.BlockSpec((din, dout), lambda m: (0, 0)),
        ],
        out_specs=[
            pl.BlockSpec((blk, dcat), lambda m: (m, 0)),
            pl.BlockSpec((blk, dout), lambda m: (m, 0)),
        ],
        out_shape=[
            jax.ShapeDtypeStruct((n, dcat), jnp.int32),
            jax.ShapeDtypeStruct((n, dout), jnp.float32),
        ],
    )(h, bcat_lo, bcat_hi, wself)


def _tc_mid(aggp, degp, selfp, bias, bcat_lo, bcat_hi, wself, blk=400):
    n, din = selfp.shape
    dcat = bcat_lo.shape[1]
    dout = wself.shape[1]

    def body(agg_ref, deg_ref, self_ref, bias_ref, blo_ref, bhi_ref, ws_ref,
             hbp_ref, self_out_ref):
        agg = agg_ref[0] + agg_ref[1]
        deg = jnp.maximum(deg_ref[0, :, 0:1] + deg_ref[1, :, 0:1], 1.0)
        pre = agg / deg + self_ref[...] + bias_ref[...]
        h1 = jnp.where(pre > 0, pre, 0.2 * pre)
        lo = jnp.dot(h1, blo_ref[...], preferred_element_type=jnp.float32)
        hi = jnp.dot(h1, bhi_ref[...], preferred_element_type=jnp.float32)
        hbp_ref[...] = _pack_bf16(lo, hi)
        self_out_ref[...] = jnp.dot(h1, ws_ref[...], preferred_element_type=jnp.float32)

    return pl.pallas_call(
        body,
        grid=(n // blk,),
        in_specs=[
            pl.BlockSpec((2, blk, din), lambda m: (0, m, 0)),
            pl.BlockSpec((2, blk, 128), lambda m: (0, m, 0)),
            pl.BlockSpec((blk, din), lambda m: (m, 0)),
            pl.BlockSpec((1, din), lambda m: (0, 0)),
            pl.BlockSpec((din, dcat), lambda m: (0, 0)),
            pl.BlockSpec((din, dcat), lambda m: (0, 0)),
            pl.BlockSpec((din, dout), lambda m: (0, 0)),
        ],
        out_specs=[
            pl.BlockSpec((blk, dcat), lambda m: (m, 0)),
            pl.BlockSpec((blk, dout), lambda m: (m, 0)),
        ],
        out_shape=[
            jax.ShapeDtypeStruct((n, dcat), jnp.int32),
            jax.ShapeDtypeStruct((n, dout), jnp.float32),
        ],
    )(aggp, degp, selfp, bias, bcat_lo, bcat_hi, wself)


def _tc_final(aggp, degp, selfp, bias, blk=400):
    n, dout = selfp.shape

    def body(agg_ref, deg_ref, self_ref, bias_ref, out_ref):
        agg = agg_ref[0] + agg_ref[1]
        deg = jnp.maximum(deg_ref[0, :, 0:1] + deg_ref[1, :, 0:1], 1.0)
        out_ref[...] = agg / deg + self_ref[...] + bias_ref[...]

    return pl.pallas_call(
        body,
        grid=(n // blk,),
        in_specs=[
            pl.BlockSpec((2, blk, dout), lambda m: (0, m, 0)),
            pl.BlockSpec((2, blk, 128), lambda m: (0, m, 0)),
            pl.BlockSpec((blk, dout), lambda m: (m, 0)),
            pl.BlockSpec((1, dout), lambda m: (0, 0)),
        ],
        out_specs=pl.BlockSpec((blk, dout), lambda m: (m, 0)),
        out_shape=jax.ShapeDtypeStruct((n, dout), jnp.float32),
    )(aggp, degp, selfp, bias)


@functools.lru_cache(maxsize=None)
def _make_sc_edge(n_pad, nblk, nb, dout):
    info = plsc.get_sparse_core_info()
    nc, ns = info.num_cores, info.num_subcores
    mesh = plsc.VectorSubcoreMesh(core_axis_name="c", subcore_axis_name="s")
    rows_t = n_pad // ns
    nhw = nb * dout // 2

    scratch = [
        pltpu.VMEM((STAGE, CH), jnp.int32),
        pltpu.VMEM((STAGE, CH), jnp.int32),
        pltpu.VMEM((STAGE, CH), jnp.int32),
        pltpu.VMEM((CH, nhw), jnp.int32),
        pltpu.VMEM((CH, nhw), jnp.int32),
        pltpu.VMEM((CH, 128), jnp.float32),
        pltpu.VMEM((CH, 128), jnp.float32),
        pltpu.VMEM((CH, dout), jnp.float32),
        pltpu.VMEM((CH, dout), jnp.float32),
        pltpu.VMEM_SHARED((n_pad, dout), jnp.float32),
    ] + [pltpu.SemaphoreType.DMA] * 6

    def body(src_hbm, rel_hbm, dst_hbm, hbp_hbm, coeff_hbm, zero_hbm, agg_out,
             srcv, relv, dstv, rows0, rows1, cbuf0, cbuf1, msg0, msg1, aggsh,
             gr0, gr1, gc0, gc1, ss0, ss1):
        rows_ = [rows0, rows1]
        cbuf_ = [cbuf0, cbuf1]
        msg_ = [msg0, msg1]
        gr = [gr0, gr1]
        gc = [gc0, gc1]
        ss = [ss0, ss1]
        cid = lax.axis_index("c")
        sid = lax.axis_index("s")
        wid = sid * nc + cid
        r0 = sid * rows_t
        pltpu.sync_copy(zero_hbm.at[pl.ds(r0, rows_t)], aggsh.at[pl.ds(r0, rows_t)])
        plsc.subcore_barrier()

        def fire_gather(c, b):
            pltpu.async_copy(coeff_hbm.at[relv.at[c]], cbuf_[b], gc[b])
            pltpu.async_copy(hbp_hbm.at[srcv.at[c]], rows_[b], gr[b])

        def wait_gather(b):
            pltpu.make_async_copy(coeff_hbm.at[relv.at[0]], cbuf_[b], gc[b]).wait()
            pltpu.make_async_copy(hbp_hbm.at[srcv.at[0]], rows_[b], gr[b]).wait()

        def compute(c, b):
            def edge_body(i, carry2):
                accs = [jnp.zeros((LANES,), jnp.float32) for _ in range(8)]
                cvec = cbuf_[b][i, pl.ds(0, LANES)]
                for bb in range(nb):
                    cb = cvec[bb]
                    for j in range(4):
                        v = rows_[b][i, pl.ds(bb * 64 + j * 16, 16)]
                        lo = lax.bitcast_convert_type(v << 16, jnp.float32)
                        hi = lax.bitcast_convert_type(v & jnp.int32(-65536), jnp.float32)
                        accs[j] = accs[j] + cb * lo
                        accs[4 + j] = accs[4 + j] + cb * hi
                for j in range(8):
                    msg_[b][i, pl.ds(j * LANES, LANES)] = accs[j]
                return carry2

            lax.fori_loop(0, CH, edge_body, 0)

        def fire_scatter(c, b):
            pltpu.async_copy(msg_[b], aggsh.at[dstv.at[c]], ss[b], add=True)

        def wait_scatter(b):
            pltpu.make_async_copy(msg_[b], aggsh.at[dstv.at[0]], ss[b]).wait()

        def block_body(bi, carry0):
            pltpu.sync_copy(src_hbm.at[wid, bi], srcv)
            pltpu.sync_copy(rel_hbm.at[wid, bi], relv)
            pltpu.sync_copy(dst_hbm.at[wid, bi], dstv)
            fire_gather(0, 0)

            def pair(k, carry):
                c0 = 2 * k
                fire_gather(c0 + 1, 1)
                wait_gather(0)

                @pl.when(k > 0)
                def _():
                    wait_scatter(0)

                compute(c0, 0)
                fire_scatter(c0, 0)

                fire_gather(c0 + 2, 0)
                wait_gather(1)

                @pl.when(k > 0)
                def _():
                    wait_scatter(1)

                compute(c0 + 1, 1)
                fire_scatter(c0 + 1, 1)
                return carry

            lax.fori_loop(0, (STAGE - 1) // 2, pair, 0)
            wait_gather(0)
            wait_scatter(0)
            compute(STAGE - 1, 0)
            fire_scatter(STAGE - 1, 0)
            wait_scatter(0)
            wait_scatter(1)
            return carry0

        lax.fori_loop(0, nblk, block_body, 0)
        plsc.subcore_barrier()
        pltpu.sync_copy(aggsh.at[pl.ds(r0, rows_t)],
                        agg_out.at[pl.ds(cid * n_pad + r0, rows_t)])

    return pl.kernel(body,
                     out_type=[jax.ShapeDtypeStruct((nc * n_pad, dout), jnp.float32)],
                     mesh=mesh, scratch_types=scratch)


CHD = 100
STAGED = 25


@functools.lru_cache(maxsize=None)
def _make_sc_deg(n_pad):
    info = plsc.get_sparse_core_info()
    nc, ns = info.num_cores, info.num_subcores
    mesh = plsc.VectorSubcoreMesh(core_axis_name="c", subcore_axis_name="s")
    rows_t = n_pad // ns

    scratch = [
        pltpu.VMEM((STAGED, CHD), jnp.int32),
        pltpu.VMEM((CHD, 128), jnp.float32),
        pltpu.VMEM_SHARED((n_pad, 128), jnp.float32),
        pltpu.SemaphoreType.DMA,
    ]

    def body(dst_hbm, one_hbm, zero_hbm, deg_out, dstv, onesv, degsh, sem):
        cid = lax.axis_index("c")
        sid = lax.axis_index("s")
        wid = sid * nc + cid
        r0 = sid * rows_t
        nblk = dst_hbm.shape[1]
        pltpu.sync_copy(zero_hbm.at[pl.ds(r0, rows_t)], degsh.at[pl.ds(r0, rows_t)])
        pltpu.sync_copy(one_hbm, onesv)
        plsc.subcore_barrier()

        def block_body(bi, carry0):
            pltpu.sync_copy(dst_hbm.at[wid, bi], dstv)

            def fire(c, carry):
                pltpu.async_copy(onesv, degsh.at[dstv.at[c]], sem, add=True)
                return carry

            lax.fori_loop(0, STAGED, fire, 0)

            def drain(c, carry):
                pltpu.make_async_copy(onesv, degsh.at[dstv.at[0]], sem).wait()
                return carry

            lax.fori_loop(0, STAGED, drain, 0)
            return carry0

        lax.fori_loop(0, nblk, block_body, 0)
        plsc.subcore_barrier()
        pltpu.sync_copy(degsh.at[pl.ds(r0, rows_t)],
                        deg_out.at[pl.ds(cid * n_pad + r0, rows_t)])

    return pl.kernel(body,
                     out_type=[jax.ShapeDtypeStruct((nc * n_pad, 128), jnp.float32)],
                     mesh=mesh, scratch_types=scratch)


def _rgcn(edges3, entity_embed, bases, coeff, wself, bias):
    n, din = entity_embed.shape
    nb = bases[0].shape[0]
    info = plsc.get_sparse_core_info()
    nc, ns = info.num_cores, info.num_subcores
    rows_t = -(-n // ns // 8) * 8
    n_pad = rows_t * ns

    src2, rel2, dst2, dst3 = edges3
    nblk = src2.shape[1]
    zero_hbm = jnp.zeros((n_pad, din), jnp.float32)
    one_hbm = jnp.ones((CHD, 128), jnp.float32)

    d1 = bases[0].shape[2]
    d2 = bases[1].shape[2]
    bcat0 = jnp.transpose(bases[0], (1, 0, 2)).reshape(din, nb, 2, d1 // 2)
    bcat1 = jnp.transpose(bases[1], (1, 0, 2)).reshape(d1, nb, 2, d2 // 2)
    bcat0_lo = bcat0[:, :, 0, :].reshape(din, nb * d1 // 2)
    bcat0_hi = bcat0[:, :, 1, :].reshape(din, nb * d1 // 2)
    bcat1_lo = bcat1[:, :, 0, :].reshape(d1, nb * d2 // 2)
    bcat1_hi = bcat1[:, :, 1, :].reshape(d1, nb * d2 // 2)
    cpad0 = jnp.pad(coeff[0], ((0, 0), (0, 128 - nb)))
    cpad1 = jnp.pad(coeff[1], ((0, 0), (0, 128 - nb)))

    hbp0, self0 = _tc_proj(entity_embed, bcat0_lo, bcat0_hi, wself[0])
    (degf,) = _make_sc_deg(n_pad)(dst3, one_hbm, zero_hbm)
    degp = degf.reshape(nc, n_pad, 128)
    sc_edge = _make_sc_edge(n_pad, nblk, nb, d1)
    (agg0f,) = sc_edge(src2, rel2, dst2, hbp0, cpad0, zero_hbm)
    agg0 = agg0f.reshape(nc, n_pad, d1)
    hbp1, self1 = _tc_mid(agg0, degp, self0, bias[0].reshape(1, -1),
                          bcat1_lo, bcat1_hi, wself[1])
    (agg1f,) = sc_edge(src2, rel2, dst2, hbp1, cpad1, zero_hbm)
    agg1 = agg1f.reshape(nc, n_pad, d2)
    return _tc_final(agg1, degp, self1, bias[1].reshape(1, -1))


def kernel(edges, entity_embed, bases0, coeff0, wself0, bias0,
           bases1, coeff1, wself1, bias1):
    e = edges.shape[0]
    info = plsc.get_sparse_core_info()
    nw = info.num_cores * info.num_subcores
    nblk = e // (nw * CH * STAGE)
    src2 = edges[:, 0].reshape(nw, nblk, STAGE, CH)
    rel2 = edges[:, 1].reshape(nw, nblk, STAGE, CH)
    dst2 = edges[:, 2].reshape(nw, nblk, STAGE, CH)
    nblkd = e // (nw * CHD * STAGED)
    dst3 = edges[:, 2].reshape(nw, nblkd, STAGED, CHD)
    return _rgcn((src2, rel2, dst2, dst3), entity_embed,
                 (bases0, bases1), (coeff0, coeff1),
                 (wself0, wself1), (bias0, bias1))


# speedup vs baseline: 11.5563x; 1.0274x over previous
import functools

import jax
import jax.numpy as jnp
from jax import lax
from jax.experimental import pallas as pl
from jax.experimental.pallas import tpu as pltpu
from jax.experimental.pallas import tpu_sc as plsc

CH = 16
STAGE = 25
LANES = 16


def _pack_bf16(lo, hi):
    lou = lax.bitcast_convert_type(lo, jnp.uint32)
    hiu = lax.bitcast_convert_type(hi, jnp.uint32)
    packed = ((hiu + jnp.uint32(0x8000)) & jnp.uint32(0xFFFF0000)) | (
        (lou + jnp.uint32(0x8000)) >> 16)
    return lax.bitcast_convert_type(packed, jnp.int32)


def _tc_proj(h, bcat_lo, bcat_hi, wself, blk=400):
    n, din = h.shape
    dcat = bcat_lo.shape[1]
    dout = wself.shape[1]

    def body(h_ref, blo_ref, bhi_ref, ws_ref, hbp_ref, self_ref):
        hblk = h_ref[...]
        lo = jnp.dot(hblk, blo_ref[...], preferred_element_type=jnp.float32)
        hi = jnp.dot(hblk, bhi_ref[...], preferred_element_type=jnp.float32)
        hbp_ref[...] = _pack_bf16(lo, hi)
        self_ref[...] = jnp.dot(hblk, ws_ref[...], preferred_element_type=jnp.float32)

    return pl.pallas_call(
        body,
        grid=(n // blk,),
        in_specs=[
            pl.BlockSpec((blk, din), lambda m: (m, 0)),
            pl.BlockSpec((din, dcat), lambda m: (0, 0)),
            pl.BlockSpec((din, dcat), lambda m: (0, 0)),
            pl.BlockSpec((din, dout), lambda m: (0, 0)),
        ],
        out_specs=[
            pl.BlockSpec((blk, dcat), lambda m: (m, 0)),
            pl.BlockSpec((blk, dout), lambda m: (m, 0)),
        ],
        out_shape=[
            jax.ShapeDtypeStruct((n, dcat), jnp.int32),
            jax.ShapeDtypeStruct((n, dout), jnp.float32),
        ],
    )(h, bcat_lo, bcat_hi, wself)


def _tc_mid(aggp, degp, selfp, bias, bcat_lo, bcat_hi, wself, blk=400):
    n, din = selfp.shape
    dcat = bcat_lo.shape[1]
    dout = wself.shape[1]

    def body(agg_ref, deg_ref, self_ref, bias_ref, blo_ref, bhi_ref, ws_ref,
             hbp_ref, self_out_ref):
        agg = agg_ref[0] + agg_ref[1]
        deg = jnp.maximum(deg_ref[0, :, 0:1] + deg_ref[1, :, 0:1], 1.0)
        pre = agg / deg + self_ref[...] + bias_ref[...]
        h1 = jnp.where(pre > 0, pre, 0.2 * pre)
        lo = jnp.dot(h1, blo_ref[...], preferred_element_type=jnp.float32)
        hi = jnp.dot(h1, bhi_ref[...], preferred_element_type=jnp.float32)
        hbp_ref[...] = _pack_bf16(lo, hi)
        self_out_ref[...] = jnp.dot(h1, ws_ref[...], preferred_element_type=jnp.float32)

    return pl.pallas_call(
        body,
        grid=(n // blk,),
        in_specs=[
            pl.BlockSpec((2, blk, din), lambda m: (0, m, 0)),
            pl.BlockSpec((2, blk, 128), lambda m: (0, m, 0)),
            pl.BlockSpec((blk, din), lambda m: (m, 0)),
            pl.BlockSpec((1, din), lambda m: (0, 0)),
            pl.BlockSpec((din, dcat), lambda m: (0, 0)),
            pl.BlockSpec((din, dcat), lambda m: (0, 0)),
            pl.BlockSpec((din, dout), lambda m: (0, 0)),
        ],
        out_specs=[
            pl.BlockSpec((blk, dcat), lambda m: (m, 0)),
            pl.BlockSpec((blk, dout), lambda m: (m, 0)),
        ],
        out_shape=[
            jax.ShapeDtypeStruct((n, dcat), jnp.int32),
            jax.ShapeDtypeStruct((n, dout), jnp.float32),
        ],
    )(aggp, degp, selfp, bias, bcat_lo, bcat_hi, wself)


def _tc_final(aggp, degp, selfp, bias, blk=400):
    n, dout = selfp.shape

    def body(agg_ref, deg_ref, self_ref, bias_ref, out_ref):
        agg = agg_ref[0] + agg_ref[1]
        deg = jnp.maximum(deg_ref[0, :, 0:1] + deg_ref[1, :, 0:1], 1.0)
        out_ref[...] = agg / deg + self_ref[...] + bias_ref[...]

    return pl.pallas_call(
        body,
        grid=(n // blk,),
        in_specs=[
            pl.BlockSpec((2, blk, dout), lambda m: (0, m, 0)),
            pl.BlockSpec((2, blk, 128), lambda m: (0, m, 0)),
            pl.BlockSpec((blk, dout), lambda m: (m, 0)),
            pl.BlockSpec((1, dout), lambda m: (0, 0)),
        ],
        out_specs=pl.BlockSpec((blk, dout), lambda m: (m, 0)),
        out_shape=jax.ShapeDtypeStruct((n, dout), jnp.float32),
    )(aggp, degp, selfp, bias)


@functools.lru_cache(maxsize=None)
def _make_sc_edge(n_pad, nblk, nb, dout):
    info = plsc.get_sparse_core_info()
    nc, ns = info.num_cores, info.num_subcores
    mesh = plsc.VectorSubcoreMesh(core_axis_name="c", subcore_axis_name="s")
    rows_t = n_pad // ns
    nhw = nb * dout // 2

    scratch = [
        pltpu.VMEM((STAGE, CH), jnp.int32),
        pltpu.VMEM((STAGE, CH), jnp.int32),
        pltpu.VMEM((STAGE, CH), jnp.int32),
        pltpu.VMEM((CH, nhw), jnp.int32),
        pltpu.VMEM((CH, nhw), jnp.int32),
        pltpu.VMEM((CH, LANES), jnp.float32),
        pltpu.VMEM((CH, LANES), jnp.float32),
        pltpu.VMEM((CH, dout), jnp.float32),
        pltpu.VMEM((CH, dout), jnp.float32),
        pltpu.VMEM_SHARED((n_pad, dout), jnp.float32),
    ] + [pltpu.SemaphoreType.DMA] * 6

    def body(src_hbm, rel_hbm, dst_hbm, hbp_hbm, coeff_hbm, zero_hbm, agg_out,
             srcv, relv, dstv, rows0, rows1, cbuf0, cbuf1, msg0, msg1, aggsh,
             gr0, gr1, gc0, gc1, ss0, ss1):
        rows_ = [rows0, rows1]
        cbuf_ = [cbuf0, cbuf1]
        msg_ = [msg0, msg1]
        gr = [gr0, gr1]
        gc = [gc0, gc1]
        ss = [ss0, ss1]
        cid = lax.axis_index("c")
        sid = lax.axis_index("s")
        wid = sid * nc + cid
        r0 = sid * rows_t
        pltpu.sync_copy(zero_hbm.at[pl.ds(r0, rows_t)], aggsh.at[pl.ds(r0, rows_t)])
        plsc.subcore_barrier()

        def fire_gather(c, b):
            pltpu.async_copy(coeff_hbm.at[relv.at[c]], cbuf_[b], gc[b])
            pltpu.async_copy(hbp_hbm.at[srcv.at[c]], rows_[b], gr[b])

        def wait_gather(b):
            pltpu.make_async_copy(coeff_hbm.at[relv.at[0]], cbuf_[b], gc[b]).wait()
            pltpu.make_async_copy(hbp_hbm.at[srcv.at[0]], rows_[b], gr[b]).wait()

        def compute(c, b):
            def edge_body(i, carry2):
                accs = [jnp.zeros((LANES,), jnp.float32) for _ in range(8)]
                cvec = cbuf_[b][i, pl.ds(0, LANES)]
                for bb in range(nb):
                    cb = cvec[bb]
                    for j in range(4):
                        v = rows_[b][i, pl.ds(bb * 64 + j * 16, 16)]
                        lo = lax.bitcast_convert_type(v << 16, jnp.float32)
                        hi = lax.bitcast_convert_type(v & jnp.int32(-65536), jnp.float32)
                        accs[j] = accs[j] + cb * lo
                        accs[4 + j] = accs[4 + j] + cb * hi
                for j in range(8):
                    msg_[b][i, pl.ds(j * LANES, LANES)] = accs[j]
                return carry2

            lax.fori_loop(0, CH, edge_body, 0)

        def fire_scatter(c, b):
            pltpu.async_copy(msg_[b], aggsh.at[dstv.at[c]], ss[b], add=True)

        def wait_scatter(b):
            pltpu.make_async_copy(msg_[b], aggsh.at[dstv.at[0]], ss[b]).wait()

        def block_body(bi, carry0):
            pltpu.sync_copy(src_hbm.at[wid, bi], srcv)
            pltpu.sync_copy(rel_hbm.at[wid, bi], relv)
            pltpu.sync_copy(dst_hbm.at[wid, bi], dstv)
            fire_gather(0, 0)

            def pair(k, carry):
                c0 = 2 * k
                fire_gather(c0 + 1, 1)
                wait_gather(0)

                @pl.when(k > 0)
                def _():
                    wait_scatter(0)

                compute(c0, 0)
                fire_scatter(c0, 0)

                fire_gather(c0 + 2, 0)
                wait_gather(1)

                @pl.when(k > 0)
                def _():
                    wait_scatter(1)

                compute(c0 + 1, 1)
                fire_scatter(c0 + 1, 1)
                return carry

            lax.fori_loop(0, (STAGE - 1) // 2, pair, 0)
            wait_gather(0)
            wait_scatter(0)
            compute(STAGE - 1, 0)
            fire_scatter(STAGE - 1, 0)
            wait_scatter(0)
            wait_scatter(1)
            return carry0

        lax.fori_loop(0, nblk, block_body, 0)
        plsc.subcore_barrier()
        pltpu.sync_copy(aggsh.at[pl.ds(r0, rows_t)],
                        agg_out.at[pl.ds(cid * n_pad + r0, rows_t)])

    return pl.kernel(body,
                     out_type=[jax.ShapeDtypeStruct((nc * n_pad, dout), jnp.float32)],
                     mesh=mesh, scratch_types=scratch,
                     compiler_params=pltpu.CompilerParams(use_tc_tiling_on_sc=False))


CHD = 100
STAGED = 25


@functools.lru_cache(maxsize=None)
def _make_sc_deg(n_pad):
    info = plsc.get_sparse_core_info()
    nc, ns = info.num_cores, info.num_subcores
    mesh = plsc.VectorSubcoreMesh(core_axis_name="c", subcore_axis_name="s")
    rows_t = n_pad // ns

    scratch = [
        pltpu.VMEM((STAGED, CHD), jnp.int32),
        pltpu.VMEM((CHD, 128), jnp.float32),
        pltpu.VMEM_SHARED((n_pad, 128), jnp.float32),
        pltpu.SemaphoreType.DMA,
    ]

    def body(dst_hbm, one_hbm, zero_hbm, deg_out, dstv, onesv, degsh, sem):
        cid = lax.axis_index("c")
        sid = lax.axis_index("s")
        wid = sid * nc + cid
        r0 = sid * rows_t
        nblk = dst_hbm.shape[1]
        pltpu.sync_copy(zero_hbm.at[pl.ds(r0, rows_t)], degsh.at[pl.ds(r0, rows_t)])
        pltpu.sync_copy(one_hbm, onesv)
        plsc.subcore_barrier()

        def block_body(bi, carry0):
            pltpu.sync_copy(dst_hbm.at[wid, bi], dstv)

            def fire(c, carry):
                pltpu.async_copy(onesv, degsh.at[dstv.at[c]], sem, add=True)
                return carry

            lax.fori_loop(0, STAGED, fire, 0)

            def drain(c, carry):
                pltpu.make_async_copy(onesv, degsh.at[dstv.at[0]], sem).wait()
                return carry

            lax.fori_loop(0, STAGED, drain, 0)
            return carry0

        lax.fori_loop(0, nblk, block_body, 0)
        plsc.subcore_barrier()
        pltpu.sync_copy(degsh.at[pl.ds(r0, rows_t)],
                        deg_out.at[pl.ds(cid * n_pad + r0, rows_t)])

    return pl.kernel(body,
                     out_type=[jax.ShapeDtypeStruct((nc * n_pad, 128), jnp.float32)],
                     mesh=mesh, scratch_types=scratch)


def _rgcn(edges3, entity_embed, bases, coeff, wself, bias):
    n, din = entity_embed.shape
    nb = bases[0].shape[0]
    info = plsc.get_sparse_core_info()
    nc, ns = info.num_cores, info.num_subcores
    rows_t = -(-n // ns // 8) * 8
    n_pad = rows_t * ns

    src2, rel2, dst2, dst3 = edges3
    nblk = src2.shape[1]
    zero_hbm = jnp.zeros((n_pad, din), jnp.float32)
    one_hbm = jnp.ones((CHD, 128), jnp.float32)

    d1 = bases[0].shape[2]
    d2 = bases[1].shape[2]
    bcat0 = jnp.transpose(bases[0], (1, 0, 2)).reshape(din, nb, 2, d1 // 2)
    bcat1 = jnp.transpose(bases[1], (1, 0, 2)).reshape(d1, nb, 2, d2 // 2)
    bcat0_lo = bcat0[:, :, 0, :].reshape(din, nb * d1 // 2)
    bcat0_hi = bcat0[:, :, 1, :].reshape(din, nb * d1 // 2)
    bcat1_lo = bcat1[:, :, 0, :].reshape(d1, nb * d2 // 2)
    bcat1_hi = bcat1[:, :, 1, :].reshape(d1, nb * d2 // 2)
    cpad0 = jnp.pad(coeff[0], ((0, 0), (0, LANES - nb)))
    cpad1 = jnp.pad(coeff[1], ((0, 0), (0, LANES - nb)))

    hbp0, self0 = _tc_proj(entity_embed, bcat0_lo, bcat0_hi, wself[0])
    (degf,) = _make_sc_deg(n_pad)(dst3, one_hbm, zero_hbm)
    degp = degf.reshape(nc, n_pad, 128)
    sc_edge = _make_sc_edge(n_pad, nblk, nb, d1)
    (agg0f,) = sc_edge(src2, rel2, dst2, hbp0, cpad0, zero_hbm)
    agg0 = agg0f.reshape(nc, n_pad, d1)
    hbp1, self1 = _tc_mid(agg0, degp, self0, bias[0].reshape(1, -1),
                          bcat1_lo, bcat1_hi, wself[1])
    (agg1f,) = sc_edge(src2, rel2, dst2, hbp1, cpad1, zero_hbm)
    agg1 = agg1f.reshape(nc, n_pad, d2)
    return _tc_final(agg1, degp, self1, bias[1].reshape(1, -1))


def kernel(edges, entity_embed, bases0, coeff0, wself0, bias0,
           bases1, coeff1, wself1, bias1):
    e = edges.shape[0]
    info = plsc.get_sparse_core_info()
    nw = info.num_cores * info.num_subcores
    nblk = e // (nw * CH * STAGE)
    src2 = edges[:, 0].reshape(nw, nblk, STAGE, CH)
    rel2 = edges[:, 1].reshape(nw, nblk, STAGE, CH)
    dst2 = edges[:, 2].reshape(nw, nblk, STAGE, CH)
    nblkd = e // (nw * CHD * STAGED)
    dst3 = edges[:, 2].reshape(nw, nblkd, STAGED, CHD)
    return _rgcn((src2, rel2, dst2, dst3), entity_embed,
                 (bases0, bases1), (coeff0, coeff1),
                 (wself0, wself1), (bias0, bias1))


# speedup vs baseline: 12.3451x; 1.0683x over previous
import functools

import jax
import jax.numpy as jnp
from jax import lax
from jax.experimental import pallas as pl
from jax.experimental.pallas import tpu as pltpu
from jax.experimental.pallas import tpu_sc as plsc

CH = 16
STAGE = 25
LANES = 16


def _pack_bf16(lo, hi):
    lou = lax.bitcast_convert_type(lo, jnp.uint32)
    hiu = lax.bitcast_convert_type(hi, jnp.uint32)
    packed = ((hiu + jnp.uint32(0x8000)) & jnp.uint32(0xFFFF0000)) | (
        (lou + jnp.uint32(0x8000)) >> 16)
    return lax.bitcast_convert_type(packed, jnp.int32)


def _tc_proj(h, bcat_lo, bcat_hi, wself, blk=400):
    n, din = h.shape
    dcat = bcat_lo.shape[1]
    dout = wself.shape[1]

    def body(h_ref, blo_ref, bhi_ref, ws_ref, hbp_ref, self_ref):
        hblk = h_ref[...]
        lo = jnp.dot(hblk, blo_ref[...], preferred_element_type=jnp.float32)
        hi = jnp.dot(hblk, bhi_ref[...], preferred_element_type=jnp.float32)
        hbp_ref[...] = _pack_bf16(lo, hi)
        self_ref[...] = jnp.dot(hblk, ws_ref[...], preferred_element_type=jnp.float32)

    return pl.pallas_call(
        body,
        grid=(n // blk,),
        in_specs=[
            pl.BlockSpec((blk, din), lambda m: (m, 0)),
            pl.BlockSpec((din, dcat), lambda m: (0, 0)),
            pl.BlockSpec((din, dcat), lambda m: (0, 0)),
            pl.BlockSpec((din, dout), lambda m: (0, 0)),
        ],
        out_specs=[
            pl.BlockSpec((blk, dcat), lambda m: (m, 0)),
            pl.BlockSpec((blk, dout), lambda m: (m, 0)),
        ],
        out_shape=[
            jax.ShapeDtypeStruct((n, dcat), jnp.int32),
            jax.ShapeDtypeStruct((n, dout), jnp.float32),
        ],
    )(h, bcat_lo, bcat_hi, wself)


def _tc_mid(aggp, degp, selfp, bias, bcat_lo, bcat_hi, wself, blk=400):
    n, din = selfp.shape
    dcat = bcat_lo.shape[1]
    dout = wself.shape[1]

    def body(agg_ref, deg_ref, self_ref, bias_ref, blo_ref, bhi_ref, ws_ref,
             hbp_ref, self_out_ref):
        agg = agg_ref[0] + agg_ref[1]
        deg = jnp.maximum(deg_ref[0, :, 0:1] + deg_ref[1, :, 0:1], 1.0)
        pre = agg / deg + self_ref[...] + bias_ref[...]
        h1 = jnp.where(pre > 0, pre, 0.2 * pre)
        lo = jnp.dot(h1, blo_ref[...], preferred_element_type=jnp.float32)
        hi = jnp.dot(h1, bhi_ref[...], preferred_element_type=jnp.float32)
        hbp_ref[...] = _pack_bf16(lo, hi)
        self_out_ref[...] = jnp.dot(h1, ws_ref[...], preferred_element_type=jnp.float32)

    return pl.pallas_call(
        body,
        grid=(n // blk,),
        in_specs=[
            pl.BlockSpec((2, blk, din), lambda m: (0, m, 0)),
            pl.BlockSpec((2, blk, 128), lambda m: (0, m, 0)),
            pl.BlockSpec((blk, din), lambda m: (m, 0)),
            pl.BlockSpec((1, din), lambda m: (0, 0)),
            pl.BlockSpec((din, dcat), lambda m: (0, 0)),
            pl.BlockSpec((din, dcat), lambda m: (0, 0)),
            pl.BlockSpec((din, dout), lambda m: (0, 0)),
        ],
        out_specs=[
            pl.BlockSpec((blk, dcat), lambda m: (m, 0)),
            pl.BlockSpec((blk, dout), lambda m: (m, 0)),
        ],
        out_shape=[
            jax.ShapeDtypeStruct((n, dcat), jnp.int32),
            jax.ShapeDtypeStruct((n, dout), jnp.float32),
        ],
    )(aggp, degp, selfp, bias, bcat_lo, bcat_hi, wself)


def _tc_final(aggp, degp, selfp, bias, blk=400):
    n, dout = selfp.shape

    def body(agg_ref, deg_ref, self_ref, bias_ref, out_ref):
        agg = agg_ref[0] + agg_ref[1]
        deg = jnp.maximum(deg_ref[0, :, 0:1] + deg_ref[1, :, 0:1], 1.0)
        out_ref[...] = agg / deg + self_ref[...] + bias_ref[...]

    return pl.pallas_call(
        body,
        grid=(n // blk,),
        in_specs=[
            pl.BlockSpec((2, blk, dout), lambda m: (0, m, 0)),
            pl.BlockSpec((2, blk, 128), lambda m: (0, m, 0)),
            pl.BlockSpec((blk, dout), lambda m: (m, 0)),
            pl.BlockSpec((1, dout), lambda m: (0, 0)),
        ],
        out_specs=pl.BlockSpec((blk, dout), lambda m: (m, 0)),
        out_shape=jax.ShapeDtypeStruct((n, dout), jnp.float32),
    )(aggp, degp, selfp, bias)


@functools.lru_cache(maxsize=None)
def _make_sc_edge(n_pad, nblk, nb, dout):
    info = plsc.get_sparse_core_info()
    nc, ns = info.num_cores, info.num_subcores
    mesh = plsc.VectorSubcoreMesh(core_axis_name="c", subcore_axis_name="s")
    rows_t = n_pad // ns
    nhw = nb * dout // 2

    scratch = [
        pltpu.VMEM((STAGE, CH), jnp.int32),
        pltpu.VMEM((STAGE, CH), jnp.int32),
        pltpu.VMEM((STAGE, CH), jnp.int32),
        pltpu.VMEM((CH, nhw), jnp.int32),
        pltpu.VMEM((CH, nhw), jnp.int32),
        pltpu.VMEM((CH, nhw), jnp.int32),
        pltpu.VMEM((CH, LANES), jnp.float32),
        pltpu.VMEM((CH, LANES), jnp.float32),
        pltpu.VMEM((CH, LANES), jnp.float32),
        pltpu.VMEM((CH, dout), jnp.float32),
        pltpu.VMEM((CH, dout), jnp.float32),
        pltpu.VMEM((CH, dout), jnp.float32),
        pltpu.VMEM_SHARED((n_pad, dout), jnp.float32),
    ] + [pltpu.SemaphoreType.DMA] * 9

    def body(src_hbm, rel_hbm, dst_hbm, hbp_hbm, coeff_hbm, zero_hbm, agg_out,
             srcv, relv, dstv, rows0, rows1, rows2, cbuf0, cbuf1, cbuf2,
             msg0, msg1, msg2, aggsh,
             gr0, gr1, gr2, gc0, gc1, gc2, ss0, ss1, ss2):
        rows_ = [rows0, rows1, rows2]
        cbuf_ = [cbuf0, cbuf1, cbuf2]
        msg_ = [msg0, msg1, msg2]
        gr = [gr0, gr1, gr2]
        gc = [gc0, gc1, gc2]
        ss = [ss0, ss1, ss2]
        cid = lax.axis_index("c")
        sid = lax.axis_index("s")
        wid = sid * nc + cid
        r0 = sid * rows_t
        pltpu.sync_copy(zero_hbm.at[pl.ds(r0, rows_t)], aggsh.at[pl.ds(r0, rows_t)])
        plsc.subcore_barrier()

        def fire_gather(c, b):
            pltpu.async_copy(coeff_hbm.at[relv.at[c]], cbuf_[b], gc[b])
            pltpu.async_copy(hbp_hbm.at[srcv.at[c]], rows_[b], gr[b])

        def wait_gather(b):
            pltpu.make_async_copy(coeff_hbm.at[relv.at[0]], cbuf_[b], gc[b]).wait()
            pltpu.make_async_copy(hbp_hbm.at[srcv.at[0]], rows_[b], gr[b]).wait()

        def compute(c, b):
            def edge_body(i, carry2):
                accs = [jnp.zeros((LANES,), jnp.float32) for _ in range(8)]
                cvec = cbuf_[b][i, pl.ds(0, LANES)]
                for bb in range(nb):
                    cb = cvec[bb]
                    for j in range(4):
                        v = rows_[b][i, pl.ds(bb * 64 + j * 16, 16)]
                        lo = lax.bitcast_convert_type(v << 16, jnp.float32)
                        hi = lax.bitcast_convert_type(v & jnp.int32(-65536), jnp.float32)
                        accs[j] = accs[j] + cb * lo
                        accs[4 + j] = accs[4 + j] + cb * hi
                for j in range(8):
                    msg_[b][i, pl.ds(j * LANES, LANES)] = accs[j]
                return carry2

            lax.fori_loop(0, CH, edge_body, 0)

        def fire_scatter(c, b):
            pltpu.async_copy(msg_[b], aggsh.at[dstv.at[c]], ss[b], add=True)

        def wait_scatter(b):
            pltpu.make_async_copy(msg_[b], aggsh.at[dstv.at[0]], ss[b]).wait()

        def block_body(bi, carry0):
            pltpu.sync_copy(src_hbm.at[wid, bi], srcv)
            pltpu.sync_copy(rel_hbm.at[wid, bi], relv)
            pltpu.sync_copy(dst_hbm.at[wid, bi], dstv)
            fire_gather(0, 0)
            fire_gather(1, 1)

            def triple(k, carry):
                for j in range(3):
                    cc = 3 * k + j
                    nxt = cc + 2

                    @pl.when(nxt < STAGE)
                    def _():
                        fire_gather(nxt, (j + 2) % 3)

                    wait_gather(j)

                    @pl.when(k > 0)
                    def _():
                        wait_scatter(j)

                    compute(cc, j)
                    fire_scatter(cc, j)
                return carry

            lax.fori_loop(0, (STAGE - 1) // 3, triple, 0)
            wait_gather((STAGE - 1) % 3)
            wait_scatter((STAGE - 1) % 3)
            compute(STAGE - 1, (STAGE - 1) % 3)
            fire_scatter(STAGE - 1, (STAGE - 1) % 3)
            wait_scatter(0)
            wait_scatter(1)
            wait_scatter(2)
            return carry0

        lax.fori_loop(0, nblk, block_body, 0)
        plsc.subcore_barrier()
        pltpu.sync_copy(aggsh.at[pl.ds(r0, rows_t)],
                        agg_out.at[pl.ds(cid * n_pad + r0, rows_t)])

    return pl.kernel(body,
                     out_type=[jax.ShapeDtypeStruct((nc * n_pad, dout), jnp.float32)],
                     mesh=mesh, scratch_types=scratch,
                     compiler_params=pltpu.CompilerParams(use_tc_tiling_on_sc=False))


CHD = 100
STAGED = 25


@functools.lru_cache(maxsize=None)
def _make_sc_deg(n_pad):
    info = plsc.get_sparse_core_info()
    nc, ns = info.num_cores, info.num_subcores
    mesh = plsc.VectorSubcoreMesh(core_axis_name="c", subcore_axis_name="s")
    rows_t = n_pad // ns

    scratch = [
        pltpu.VMEM((STAGED, CHD), jnp.int32),
        pltpu.VMEM((CHD, 128), jnp.float32),
        pltpu.VMEM_SHARED((n_pad, 128), jnp.float32),
        pltpu.SemaphoreType.DMA,
    ]

    def body(dst_hbm, one_hbm, zero_hbm, deg_out, dstv, onesv, degsh, sem):
        cid = lax.axis_index("c")
        sid = lax.axis_index("s")
        wid = sid * nc + cid
        r0 = sid * rows_t
        nblk = dst_hbm.shape[1]
        pltpu.sync_copy(zero_hbm.at[pl.ds(r0, rows_t)], degsh.at[pl.ds(r0, rows_t)])
        pltpu.sync_copy(one_hbm, onesv)
        plsc.subcore_barrier()

        def block_body(bi, carry0):
            pltpu.sync_copy(dst_hbm.at[wid, bi], dstv)

            def fire(c, carry):
                pltpu.async_copy(onesv, degsh.at[dstv.at[c]], sem, add=True)
                return carry

            lax.fori_loop(0, STAGED, fire, 0)

            def drain(c, carry):
                pltpu.make_async_copy(onesv, degsh.at[dstv.at[0]], sem).wait()
                return carry

            lax.fori_loop(0, STAGED, drain, 0)
            return carry0

        lax.fori_loop(0, nblk, block_body, 0)
        plsc.subcore_barrier()
        pltpu.sync_copy(degsh.at[pl.ds(r0, rows_t)],
                        deg_out.at[pl.ds(cid * n_pad + r0, rows_t)])

    return pl.kernel(body,
                     out_type=[jax.ShapeDtypeStruct((nc * n_pad, 128), jnp.float32)],
                     mesh=mesh, scratch_types=scratch)


def _rgcn(edges3, entity_embed, bases, coeff, wself, bias):
    n, din = entity_embed.shape
    nb = bases[0].shape[0]
    info = plsc.get_sparse_core_info()
    nc, ns = info.num_cores, info.num_subcores
    rows_t = -(-n // ns // 8) * 8
    n_pad = rows_t * ns

    src2, rel2, dst2, dst3 = edges3
    nblk = src2.shape[1]
    zero_hbm = jnp.zeros((n_pad, din), jnp.float32)
    one_hbm = jnp.ones((CHD, 128), jnp.float32)

    d1 = bases[0].shape[2]
    d2 = bases[1].shape[2]
    bcat0 = jnp.transpose(bases[0], (1, 0, 2)).reshape(din, nb, 2, d1 // 2)
    bcat1 = jnp.transpose(bases[1], (1, 0, 2)).reshape(d1, nb, 2, d2 // 2)
    bcat0_lo = bcat0[:, :, 0, :].reshape(din, nb * d1 // 2)
    bcat0_hi = bcat0[:, :, 1, :].reshape(din, nb * d1 // 2)
    bcat1_lo = bcat1[:, :, 0, :].reshape(d1, nb * d2 // 2)
    bcat1_hi = bcat1[:, :, 1, :].reshape(d1, nb * d2 // 2)
    cpad0 = jnp.pad(coeff[0], ((0, 0), (0, LANES - nb)))
    cpad1 = jnp.pad(coeff[1], ((0, 0), (0, LANES - nb)))

    hbp0, self0 = _tc_proj(entity_embed, bcat0_lo, bcat0_hi, wself[0])
    (degf,) = _make_sc_deg(n_pad)(dst3, one_hbm, zero_hbm)
    degp = degf.reshape(nc, n_pad, 128)
    sc_edge = _make_sc_edge(n_pad, nblk, nb, d1)
    (agg0f,) = sc_edge(src2, rel2, dst2, hbp0, cpad0, zero_hbm)
    agg0 = agg0f.reshape(nc, n_pad, d1)
    hbp1, self1 = _tc_mid(agg0, degp, self0, bias[0].reshape(1, -1),
                          bcat1_lo, bcat1_hi, wself[1])
    (agg1f,) = sc_edge(src2, rel2, dst2, hbp1, cpad1, zero_hbm)
    agg1 = agg1f.reshape(nc, n_pad, d2)
    return _tc_final(agg1, degp, self1, bias[1].reshape(1, -1))


def kernel(edges, entity_embed, bases0, coeff0, wself0, bias0,
           bases1, coeff1, wself1, bias1):
    e = edges.shape[0]
    info = plsc.get_sparse_core_info()
    nw = info.num_cores * info.num_subcores
    nblk = e // (nw * CH * STAGE)
    src2 = edges[:, 0].reshape(nw, nblk, STAGE, CH)
    rel2 = edges[:, 1].reshape(nw, nblk, STAGE, CH)
    dst2 = edges[:, 2].reshape(nw, nblk, STAGE, CH)
    nblkd = e // (nw * CHD * STAGED)
    dst3 = edges[:, 2].reshape(nw, nblkd, STAGED, CHD)
    return _rgcn((src2, rel2, dst2, dst3), entity_embed,
                 (bases0, bases1), (coeff0, coeff1),
                 (wself0, wself1), (bias0, bias1))
